# baseline (speedup 1.0000x reference)
import functools

import jax
import jax.numpy as jnp
from jax import lax
from jax.experimental import pallas as pl
from jax.experimental.pallas import tpu as pltpu
from jax.experimental.pallas import tpu_sc as plsc


def kernel(labels, train, table):
    del train
    B = labels.shape[0]
    V, D = table.shape
    info = plsc.get_sparse_core_info()
    NC, NS = info.num_cores, info.num_subcores
    NW = NC * NS
    b_per_w = B // NW

    C = 64
    n_chunks = b_per_w // C

    mesh = plsc.VectorSubcoreMesh(core_axis_name="c", subcore_axis_name="s")

    @functools.partial(
        pl.kernel,
        mesh=mesh,
        compiler_params=pltpu.CompilerParams(use_tc_tiling_on_sc=False),
        out_type=jax.ShapeDtypeStruct((B, D), jnp.float32),
        scratch_types=[
            pltpu.VMEM((n_chunks, C), jnp.int32),
            pltpu.VMEM((n_chunks, C, D), jnp.float32),
            pltpu.SemaphoreType.DMA((n_chunks,)),
            pltpu.SemaphoreType.DMA,
        ],
    )
    def emb(table_hbm, idx_hbm, out_hbm, idx_v, rows_v, gsems, psem):
        wid = lax.axis_index("s") * NC + lax.axis_index("c")
        base = wid * b_per_w
        pltpu.sync_copy(idx_hbm.at[wid], idx_v)
        gets = []
        for j in range(n_chunks):
            gets.append(
                pltpu.async_copy(
                    table_hbm.at[idx_v.at[j]],
                    rows_v.at[j],
                    gsems.at[j],
                )
            )
        puts = []
        for j in range(n_chunks):
            gets[j].wait()
            puts.append(
                pltpu.async_copy(
                    rows_v.at[j], out_hbm.at[pl.ds(base + j * C, C)], psem
                )
            )
        for p in puts:
            p.wait()

    return emb(table, labels.reshape(NW, n_chunks, C))


# speedup vs baseline: 1.6595x; 1.6595x over previous
import functools

import jax
import jax.numpy as jnp
from jax import lax
from jax.experimental import pallas as pl
from jax.experimental.pallas import tpu as pltpu
from jax.experimental.pallas import tpu_sc as plsc


def kernel(labels, train, table):
    del train
    B = labels.shape[0]
    V, D = table.shape
    info = plsc.get_sparse_core_info()
    NC, NS = info.num_cores, info.num_subcores
    NW = NC * NS
    b_per_w = B // NW

    C = 16
    NG = b_per_w // C

    mesh = plsc.VectorSubcoreMesh(core_axis_name="c", subcore_axis_name="s")

    @functools.partial(
        pl.kernel,
        mesh=mesh,
        out_type=jax.ShapeDtypeStruct((B, D), jnp.float32),
        scratch_types=[
            pltpu.VMEM((b_per_w,), jnp.int32),
            pltpu.VMEM((2, C, D), jnp.float32),
            pltpu.SemaphoreType.DMA((2,)),
            pltpu.SemaphoreType.DMA((2,)),
        ],
    )
    def emb(table_hbm, idx_hbm, out_hbm, idx_s, rows_v, gsems, psems):
        wid = lax.axis_index("s") * NC + lax.axis_index("c")
        base = wid * b_per_w
        pltpu.sync_copy(idx_hbm.at[wid], idx_s)

        def issue_group(g, buf):
            vec = idx_s[pl.ds(g * C, C)]
            cps = []
            for s in range(C):
                i = vec[s]
                cps.append(
                    pltpu.async_copy(
                        table_hbm.at[pl.ds(i, 1)],
                        rows_v.at[buf].at[pl.ds(s, 1)],
                        gsems.at[buf],
                    )
                )
            return cps

        def drain_group(cps):
            for cp in cps:
                cp.wait()

        def writeback(g, buf):
            return pltpu.async_copy(
                rows_v.at[buf], out_hbm.at[pl.ds(base + g * C, C)], psems.at[buf]
            )

        def wait_writeback(buf):
            pltpu.make_async_copy(
                rows_v.at[buf], out_hbm.at[pl.ds(base, C)], psems.at[buf]
            ).wait()

        @pl.loop(0, NG // 2)
        def _(gg):
            g0 = gg * 2
            g1 = g0 + 1

            @pl.when(gg > 0)
            def _():
                wait_writeback(0)
                wait_writeback(1)

            cps0 = issue_group(g0, 0)
            cps1 = issue_group(g1, 1)
            drain_group(cps0)
            writeback(g0, 0)
            drain_group(cps1)
            writeback(g1, 1)

        wait_writeback(0)
        wait_writeback(1)

    return emb(table, labels.reshape(NW, b_per_w))


# speedup vs baseline: 1.6786x; 1.0115x over previous
import functools

import jax
import jax.numpy as jnp
from jax import lax
from jax.experimental import pallas as pl
from jax.experimental.pallas import tpu as pltpu
from jax.experimental.pallas import tpu_sc as plsc


def kernel(labels, train, table):
    del train
    B = labels.shape[0]
    V, D = table.shape
    info = plsc.get_sparse_core_info()
    NC, NS = info.num_cores, info.num_subcores
    NW = NC * NS
    b_per_w = B // NW

    C = 32
    NG = b_per_w // C

    mesh = plsc.VectorSubcoreMesh(core_axis_name="c", subcore_axis_name="s")

    @functools.partial(
        pl.kernel,
        mesh=mesh,
        out_type=jax.ShapeDtypeStruct((B, D), jnp.float32),
        scratch_types=[
            pltpu.VMEM((b_per_w,), jnp.int32),
            pltpu.VMEM((2, C, D), jnp.float32),
            pltpu.SemaphoreType.DMA((2,)),
            pltpu.SemaphoreType.DMA((2,)),
        ],
    )
    def emb(table_hbm, idx_hbm, out_hbm, idx_s, rows_v, gsems, psems):
        wid = lax.axis_index("s") * NC + lax.axis_index("c")
        base = wid * b_per_w
        pltpu.sync_copy(idx_hbm.at[wid], idx_s)

        def issue_group(g, buf):
            cps = []
            for v in range(C // 16):
                vec = idx_s[pl.ds(g * C + v * 16, 16)]
                for s in range(16):
                    i = vec[s]
                    cps.append(
                        pltpu.async_copy(
                            table_hbm.at[pl.ds(i, 1)],
                            rows_v.at[buf].at[pl.ds(v * 16 + s, 1)],
                            gsems.at[buf],
                        )
                    )
            return cps

        def drain_group(cps):
            for cp in cps:
                cp.wait()

        def writeback(g, buf):
            return pltpu.async_copy(
                rows_v.at[buf], out_hbm.at[pl.ds(base + g * C, C)], psems.at[buf]
            )

        def wait_writeback(buf):
            pltpu.make_async_copy(
                rows_v.at[buf], out_hbm.at[pl.ds(base, C)], psems.at[buf]
            ).wait()

        @pl.loop(0, NG // 2)
        def _(gg):
            g0 = gg * 2
            g1 = g0 + 1

            @pl.when(gg > 0)
            def _():
                wait_writeback(0)
                wait_writeback(1)

            cps0 = issue_group(g0, 0)
            cps1 = issue_group(g1, 1)
            drain_group(cps0)
            writeback(g0, 0)
            drain_group(cps1)
            writeback(g1, 1)

        wait_writeback(0)
        wait_writeback(1)

    return emb(table, labels.reshape(NW, b_per_w))
